# baseline (speedup 1.0000x reference)
import functools

import jax
import jax.numpy as jnp
from jax import lax
from jax.experimental import pallas as pl
from jax.experimental.pallas import tpu as pltpu
from jax.experimental.pallas import tpu_sc as plsc

N = 10000
E = 320000
NNZ = 320000
D = 128

NC = 2
NS = 16
NW = NC * NS
EPW = E // NW
CH = 128
NCHUNK = 80
PW = NCHUNK * CH
PAD = PW - EPW
BCH = 16
NBLK = NCHUNK // BCH
WORDS_PER_TILE = N * D // NS
ZWORDS = 4000
N_PAD = 10240
RPT = N_PAD // NS

_mesh = plsc.VectorSubcoreMesh(core_axis_name="c", subcore_axis_name="s")


def _zero_fill_rows(zbuf, nrows):
    zero16 = jnp.zeros((16,), jnp.float32)

    def body(i, _):
        for q in range(D // 16):
            zbuf[i, pl.ds(q * 16, 16)] = zero16
        return 0

    lax.fori_loop(0, nrows, body, 0)


def _zero_fill_flat(zbuf, nwords):
    zero16 = jnp.zeros((16,), jnp.float32)

    def body(i, _):
        zbuf[pl.ds(i * 16, 16)] = zero16
        return 0

    lax.fori_loop(0, nwords // 16, body, 0)


def _densify_body(xr, xc, xv, out, acc, st_r, st_c, st_v, idx2, zbuf):
    c = lax.axis_index("c")
    s = lax.axis_index("s")
    wid = c * NS + s

    _zero_fill_flat(zbuf, ZWORDS)
    for q in range(WORDS_PER_TILE // ZWORDS):
        pltpu.sync_copy(
            zbuf, acc.at[pl.ds(s * WORDS_PER_TILE + q * ZWORDS, ZWORDS)])
    plsc.subcore_barrier()

    pltpu.sync_copy(xr.at[wid], st_r)
    pltpu.sync_copy(xc.at[wid], st_c)
    pltpu.sync_copy(xv.at[wid], st_v)

    def idx_body(j, _):
        for q in range(CH // 16):
            r = st_r[j, pl.ds(q * 16, 16)]
            cc = st_c[j, pl.ds(q * 16, 16)]
            idx2[j, pl.ds(q * 16, 16)] = r * D + cc
        return 0

    lax.fori_loop(0, NCHUNK, idx_body, 0)

    def scat_body(j, _):
        pltpu.sync_copy(st_v.at[j], acc.at[idx2.at[j]], add=True)
        return 0

    lax.fori_loop(0, NCHUNK, scat_body, 0)
    plsc.subcore_barrier()

    pltpu.sync_copy(acc.at[pl.ds(s * WORDS_PER_TILE, WORDS_PER_TILE)],
                    out.at[pl.ds(wid * WORDS_PER_TILE, WORDS_PER_TILE)])


_densify = functools.partial(
    pl.kernel,
    out_type=jax.ShapeDtypeStruct((NC * N * D,), jnp.float32),
    mesh=_mesh,
    scratch_types=[
        pltpu.VMEM_SHARED((N * D,), jnp.float32),
        pltpu.VMEM((NCHUNK, CH), jnp.int32),
        pltpu.VMEM((NCHUNK, CH), jnp.int32),
        pltpu.VMEM((NCHUNK, CH), jnp.float32),
        pltpu.VMEM((NCHUNK, CH), jnp.int32),
        pltpu.VMEM((ZWORDS,), jnp.float32),
    ],
)(_densify_body)


def _propagate_body(xw, ar, ac, av, out, acc, st_r, st_c, st_v, rbuf0, rbuf1,
                    gsem0, gsem1, ssem0, ssem1):
    c = lax.axis_index("c")
    s = lax.axis_index("s")
    wid = c * NS + s

    _zero_fill_rows(rbuf0, CH)
    for q in range(RPT // CH):
        pltpu.sync_copy(rbuf0, acc.at[pl.ds(s * RPT + q * CH, CH)])
    plsc.subcore_barrier()

    def scale(buf, j):
        def row_grp(g, _):
            v = st_v[j, pl.ds(g * 16, 16)]
            for i in range(16):
                w = v[i]
                for q in range(D // 16):
                    buf[g * 16 + i, pl.ds(q * 16, 16)] = (
                        buf[g * 16 + i, pl.ds(q * 16, 16)] * w)
            return 0

        lax.fori_loop(0, CH // 16, row_grp, 0)

    for blk in range(NBLK):
        pltpu.sync_copy(ar.at[wid, pl.ds(blk * BCH, BCH)], st_r)
        pltpu.sync_copy(ac.at[wid, pl.ds(blk * BCH, BCH)], st_c)
        pltpu.sync_copy(av.at[wid, pl.ds(blk * BCH, BCH)], st_v)

        pltpu.async_copy(xw.at[st_c.at[0]], rbuf0, gsem0)
        pltpu.async_copy(xw.at[st_c.at[1]], rbuf1, gsem1)

        def pair_body(t, _):
            e = 2 * t
            o = 2 * t + 1
            pltpu.make_async_copy(xw.at[st_c.at[e]], rbuf0, gsem0).wait()
            scale(rbuf0, e)
            pltpu.async_copy(rbuf0, acc.at[st_r.at[e]], ssem0, add=True)
            pltpu.make_async_copy(xw.at[st_c.at[o]], rbuf1, gsem1).wait()
            scale(rbuf1, o)
            pltpu.async_copy(rbuf1, acc.at[st_r.at[o]], ssem1, add=True)
            pltpu.make_async_copy(rbuf0, acc.at[st_r.at[e]], ssem0).wait()

            @pl.when(t < BCH // 2 - 1)
            def _():
                pltpu.async_copy(xw.at[st_c.at[e + 2]], rbuf0, gsem0)

            pltpu.make_async_copy(rbuf1, acc.at[st_r.at[o]], ssem1).wait()

            @pl.when(t < BCH // 2 - 1)
            def _():
                pltpu.async_copy(xw.at[st_c.at[o + 2]], rbuf1, gsem1)

            return 0

        lax.fori_loop(0, BCH // 2, pair_body, 0)

    plsc.subcore_barrier()

    pltpu.sync_copy(acc.at[pl.ds(s * RPT, RPT)],
                    out.at[c, pl.ds(s * RPT, RPT)])


_propagate = functools.partial(
    pl.kernel,
    out_type=jax.ShapeDtypeStruct((NC, N_PAD, D), jnp.float32),
    mesh=_mesh,
    scratch_types=[
        pltpu.VMEM_SHARED((N_PAD, D), jnp.float32),
        pltpu.VMEM((BCH, CH), jnp.int32),
        pltpu.VMEM((BCH, CH), jnp.int32),
        pltpu.VMEM((BCH, CH), jnp.float32),
        pltpu.VMEM((CH, D), jnp.float32),
        pltpu.VMEM((CH, D), jnp.float32),
        pltpu.SemaphoreType.DMA,
        pltpu.SemaphoreType.DMA,
        pltpu.SemaphoreType.DMA,
        pltpu.SemaphoreType.DMA,
    ],
)(_propagate_body)


def _matmul_body(xd_ref, w_ref, o_ref):
    o_ref[...] = jnp.dot(xd_ref[0] + xd_ref[1], w_ref[...],
                         preferred_element_type=jnp.float32)


def _finish_body(a_ref, b_ref, o_ref):
    o_ref[...] = jnp.maximum(a_ref[0] + a_ref[1] + b_ref[...], 0.0)


_BLK = 400


def _pad_split(a, pad):
    a = jnp.concatenate([a.reshape(NW, EPW), pad.astype(a.dtype)], axis=1)
    return a.reshape(NW, NCHUNK, CH)


def kernel(x_rows, x_cols, x_vals, adj_rows, adj_cols, adj_vals, W, b):
    wj = jnp.arange(NW, dtype=jnp.int32)[:, None]
    pj = jnp.arange(PAD, dtype=jnp.int32)[None, :]
    zpad = jnp.zeros((NW, PAD), jnp.float32)
    xr = _pad_split(x_rows, (wj * 37 + pj * 53) % N)
    xc = _pad_split(x_cols, (wj * 5 + pj) % D)
    xv = _pad_split(x_vals, zpad)
    ar = _pad_split(adj_rows, N + (wj * 7 + pj) % (N_PAD - N))
    ac = _pad_split(adj_cols, (wj * 131 + pj * 97) % N)
    av = _pad_split(adj_vals, zpad)

    xd = _densify(xr, xc, xv).reshape(NC, N, D)

    xw = pl.pallas_call(
        _matmul_body,
        grid=(N // _BLK,),
        in_specs=[
            pl.BlockSpec((NC, _BLK, D), lambda i: (0, i, 0)),
            pl.BlockSpec((D, D), lambda i: (0, 0)),
        ],
        out_specs=pl.BlockSpec((_BLK, D), lambda i: (i, 0)),
        out_shape=jax.ShapeDtypeStruct((N, D), jnp.float32),
    )(xd, W)

    ax = _propagate(xw, ar, ac, av)

    out = pl.pallas_call(
        _finish_body,
        grid=(N // _BLK,),
        in_specs=[
            pl.BlockSpec((NC, _BLK, D), lambda i: (0, i, 0)),
            pl.BlockSpec((1, D), lambda i: (0, 0)),
        ],
        out_specs=pl.BlockSpec((_BLK, D), lambda i: (i, 0)),
        out_shape=jax.ShapeDtypeStruct((N, D), jnp.float32),
    )(ax, b.reshape(1, D))

    return out


# speedup vs baseline: 1.0982x; 1.0982x over previous
import functools

import jax
import jax.numpy as jnp
from jax import lax
from jax.experimental import pallas as pl
from jax.experimental.pallas import tpu as pltpu
from jax.experimental.pallas import tpu_sc as plsc

N = 10000
E = 320000
NNZ = 320000
D = 128

NC = 2
NS = 16
NW = NC * NS
EPW = E // NW
CH = 128
NCHUNK = 80
PW = NCHUNK * CH
PAD = PW - EPW
BCH = 16
NBLK = NCHUNK // BCH
WORDS_PER_TILE = N * D // NS
ZWORDS = 4000
N_PAD = 10240
RPT = N_PAD // NS

_mesh = plsc.VectorSubcoreMesh(core_axis_name="c", subcore_axis_name="s")


def _zero_fill_rows(zbuf, nrows):
    zero16 = jnp.zeros((16,), jnp.float32)

    def body(i, _):
        for q in range(D // 16):
            zbuf[i, pl.ds(q * 16, 16)] = zero16
        return 0

    lax.fori_loop(0, nrows, body, 0)


def _zero_fill_flat(zbuf, nwords):
    zero16 = jnp.zeros((16,), jnp.float32)

    def body(i, _):
        zbuf[pl.ds(i * 16, 16)] = zero16
        return 0

    lax.fori_loop(0, nwords // 16, body, 0)


def _densify_body(xr, xc, xv, out, acc, st_r, st_c, st_v, idx2, zbuf, dsem):
    c = lax.axis_index("c")
    s = lax.axis_index("s")
    wid = c * NS + s

    _zero_fill_flat(zbuf, ZWORDS)
    for q in range(WORDS_PER_TILE // ZWORDS):
        pltpu.async_copy(
            zbuf, acc.at[pl.ds(s * WORDS_PER_TILE + q * ZWORDS, ZWORDS)], dsem)

    pltpu.sync_copy(xr.at[wid], st_r)
    pltpu.sync_copy(xc.at[wid], st_c)
    pltpu.sync_copy(xv.at[wid], st_v)

    for q in range(WORDS_PER_TILE // ZWORDS):
        pltpu.make_async_copy(
            zbuf, acc.at[pl.ds(s * WORDS_PER_TILE + q * ZWORDS, ZWORDS)],
            dsem).wait()
    plsc.subcore_barrier()

    def idx_body(j, _):
        for q in range(CH // 16):
            r = st_r[j, pl.ds(q * 16, 16)]
            cc = st_c[j, pl.ds(q * 16, 16)]
            idx2[j, pl.ds(q * 16, 16)] = r * D + cc
        return 0

    lax.fori_loop(0, NCHUNK, idx_body, 0)

    def scat_body(j, _):
        pltpu.async_copy(st_v.at[j], acc.at[idx2.at[j]], dsem, add=True)
        return 0

    lax.fori_loop(0, NCHUNK, scat_body, 0)

    def drain_body(j, _):
        pltpu.make_async_copy(st_v.at[j], acc.at[idx2.at[j]], dsem).wait()
        return 0

    lax.fori_loop(0, NCHUNK, drain_body, 0)
    plsc.subcore_barrier()

    pltpu.sync_copy(acc.at[pl.ds(s * WORDS_PER_TILE, WORDS_PER_TILE)],
                    out.at[pl.ds(wid * WORDS_PER_TILE, WORDS_PER_TILE)])


_densify = functools.partial(
    pl.kernel,
    out_type=jax.ShapeDtypeStruct((NC * N * D,), jnp.float32),
    mesh=_mesh,
    scratch_types=[
        pltpu.VMEM_SHARED((N * D,), jnp.float32),
        pltpu.VMEM((NCHUNK, CH), jnp.int32),
        pltpu.VMEM((NCHUNK, CH), jnp.int32),
        pltpu.VMEM((NCHUNK, CH), jnp.float32),
        pltpu.VMEM((NCHUNK, CH), jnp.int32),
        pltpu.VMEM((ZWORDS,), jnp.float32),
        pltpu.SemaphoreType.DMA,
    ],
)(_densify_body)


def _propagate_body(xw, ar, ac, av, out, acc, st_r, st_c, st_v, rbuf0, rbuf1,
                    gsem0, gsem1, ssem0, ssem1):
    c = lax.axis_index("c")
    s = lax.axis_index("s")
    wid = c * NS + s

    _zero_fill_rows(rbuf0, CH)
    for q in range(RPT // CH):
        pltpu.sync_copy(rbuf0, acc.at[pl.ds(s * RPT + q * CH, CH)])
    plsc.subcore_barrier()

    def scale(buf, j):
        def row_grp(g, _):
            v = st_v[j, pl.ds(g * 16, 16)]
            for i in range(16):
                w = v[i]
                for q in range(D // 16):
                    buf[g * 16 + i, pl.ds(q * 16, 16)] = (
                        buf[g * 16 + i, pl.ds(q * 16, 16)] * w)
            return 0

        lax.fori_loop(0, CH // 16, row_grp, 0)

    for blk in range(NBLK):
        pltpu.sync_copy(ar.at[wid, pl.ds(blk * BCH, BCH)], st_r)
        pltpu.sync_copy(ac.at[wid, pl.ds(blk * BCH, BCH)], st_c)
        pltpu.sync_copy(av.at[wid, pl.ds(blk * BCH, BCH)], st_v)

        pltpu.async_copy(xw.at[st_c.at[0]], rbuf0, gsem0)

        def pair_body(t, _):
            e = 2 * t
            o = 2 * t + 1
            pltpu.make_async_copy(xw.at[st_c.at[e]], rbuf0, gsem0).wait()
            pltpu.async_copy(xw.at[st_c.at[o]], rbuf1, gsem1)
            scale(rbuf0, e)
            pltpu.sync_copy(rbuf0, acc.at[st_r.at[e]], add=True)

            @pl.when(t < BCH // 2 - 1)
            def _():
                pltpu.async_copy(xw.at[st_c.at[e + 2]], rbuf0, gsem0)

            pltpu.make_async_copy(xw.at[st_c.at[o]], rbuf1, gsem1).wait()
            scale(rbuf1, o)
            pltpu.sync_copy(rbuf1, acc.at[st_r.at[o]], add=True)
            return 0

        lax.fori_loop(0, BCH // 2, pair_body, 0)

    plsc.subcore_barrier()

    pltpu.sync_copy(acc.at[pl.ds(s * RPT, RPT)],
                    out.at[c, pl.ds(s * RPT, RPT)])


_propagate = functools.partial(
    pl.kernel,
    out_type=jax.ShapeDtypeStruct((NC, N_PAD, D), jnp.float32),
    mesh=_mesh,
    scratch_types=[
        pltpu.VMEM_SHARED((N_PAD, D), jnp.float32),
        pltpu.VMEM((BCH, CH), jnp.int32),
        pltpu.VMEM((BCH, CH), jnp.int32),
        pltpu.VMEM((BCH, CH), jnp.float32),
        pltpu.VMEM((CH, D), jnp.float32),
        pltpu.VMEM((CH, D), jnp.float32),
        pltpu.SemaphoreType.DMA,
        pltpu.SemaphoreType.DMA,
        pltpu.SemaphoreType.DMA,
        pltpu.SemaphoreType.DMA,
    ],
)(_propagate_body)


def _matmul_body(xd_ref, w_ref, o_ref):
    o_ref[...] = jnp.dot(xd_ref[0] + xd_ref[1], w_ref[...],
                         preferred_element_type=jnp.float32)


def _finish_body(a_ref, b_ref, o_ref):
    o_ref[...] = jnp.maximum(a_ref[0] + a_ref[1] + b_ref[...], 0.0)


_BLK = 400


def _pad_split(a, pad):
    a = jnp.concatenate([a.reshape(NW, EPW), pad.astype(a.dtype)], axis=1)
    return a.reshape(NW, NCHUNK, CH)


def kernel(x_rows, x_cols, x_vals, adj_rows, adj_cols, adj_vals, W, b):
    wj = jnp.arange(NW, dtype=jnp.int32)[:, None]
    pj = jnp.arange(PAD, dtype=jnp.int32)[None, :]
    zpad = jnp.zeros((NW, PAD), jnp.float32)
    xr = _pad_split(x_rows, (wj * 37 + pj * 53) % N)
    xc = _pad_split(x_cols, (wj * 5 + pj) % D)
    xv = _pad_split(x_vals, zpad)
    ar = _pad_split(adj_rows, N + (wj * 7 + pj) % (N_PAD - N))
    ac = _pad_split(adj_cols, (wj * 131 + pj * 97) % N)
    av = _pad_split(adj_vals, zpad)

    xd = _densify(xr, xc, xv).reshape(NC, N, D)

    xw = pl.pallas_call(
        _matmul_body,
        grid=(N // _BLK,),
        in_specs=[
            pl.BlockSpec((NC, _BLK, D), lambda i: (0, i, 0)),
            pl.BlockSpec((D, D), lambda i: (0, 0)),
        ],
        out_specs=pl.BlockSpec((_BLK, D), lambda i: (i, 0)),
        out_shape=jax.ShapeDtypeStruct((N, D), jnp.float32),
    )(xd, W)

    ax = _propagate(xw, ar, ac, av)

    out = pl.pallas_call(
        _finish_body,
        grid=(N // _BLK,),
        in_specs=[
            pl.BlockSpec((NC, _BLK, D), lambda i: (0, i, 0)),
            pl.BlockSpec((1, D), lambda i: (0, 0)),
        ],
        out_specs=pl.BlockSpec((_BLK, D), lambda i: (i, 0)),
        out_shape=jax.ShapeDtypeStruct((N, D), jnp.float32),
    )(ax, b.reshape(1, D))

    return out


# speedup vs baseline: 1.1581x; 1.0546x over previous
import functools

import jax
import jax.numpy as jnp
from jax import lax
from jax.experimental import pallas as pl
from jax.experimental.pallas import tpu as pltpu
from jax.experimental.pallas import tpu_sc as plsc

N = 10000
E = 320000
NNZ = 320000
D = 128

NC = 2
NS = 16
NW = NC * NS
EPW = E // NW
CH = 128
NCHUNK = 80
PW = NCHUNK * CH
PAD = PW - EPW
BCH = 40
NBLK = NCHUNK // BCH
WORDS_PER_TILE = N * D // NS
ZWORDS = 4000
N_PAD = 10240
RPT = N_PAD // NS

_mesh = plsc.VectorSubcoreMesh(core_axis_name="c", subcore_axis_name="s")


def _zero_fill_rows(zbuf, nrows):
    zero16 = jnp.zeros((16,), jnp.float32)

    def body(i, _):
        for q in range(D // 16):
            zbuf[i, pl.ds(q * 16, 16)] = zero16
        return 0

    lax.fori_loop(0, nrows, body, 0)


def _zero_fill_flat(zbuf, nwords):
    zero16 = jnp.zeros((16,), jnp.float32)

    def body(i, _):
        zbuf[pl.ds(i * 16, 16)] = zero16
        return 0

    lax.fori_loop(0, nwords // 16, body, 0)


def _densify_body(xr, xc, xv, out, acc, st_r, st_c, st_v, idx2, zbuf, dsem):
    c = lax.axis_index("c")
    s = lax.axis_index("s")
    wid = c * NS + s

    _zero_fill_flat(zbuf, ZWORDS)
    for q in range(WORDS_PER_TILE // ZWORDS):
        pltpu.async_copy(
            zbuf, acc.at[pl.ds(s * WORDS_PER_TILE + q * ZWORDS, ZWORDS)], dsem)

    pltpu.sync_copy(xr.at[wid], st_r)
    pltpu.sync_copy(xc.at[wid], st_c)
    pltpu.sync_copy(xv.at[wid], st_v)

    for q in range(WORDS_PER_TILE // ZWORDS):
        pltpu.make_async_copy(
            zbuf, acc.at[pl.ds(s * WORDS_PER_TILE + q * ZWORDS, ZWORDS)],
            dsem).wait()
    plsc.subcore_barrier()

    def idx_body(j, _):
        for q in range(CH // 16):
            r = st_r[j, pl.ds(q * 16, 16)]
            cc = st_c[j, pl.ds(q * 16, 16)]
            idx2[j, pl.ds(q * 16, 16)] = r * D + cc
        return 0

    lax.fori_loop(0, NCHUNK, idx_body, 0)

    def scat_body(j, _):
        pltpu.async_copy(st_v.at[j], acc.at[idx2.at[j]], dsem, add=True)
        return 0

    lax.fori_loop(0, NCHUNK, scat_body, 0)

    def drain_body(j, _):
        pltpu.make_async_copy(st_v.at[j], acc.at[idx2.at[j]], dsem).wait()
        return 0

    lax.fori_loop(0, NCHUNK, drain_body, 0)
    plsc.subcore_barrier()

    pltpu.sync_copy(acc.at[pl.ds(s * WORDS_PER_TILE, WORDS_PER_TILE)],
                    out.at[pl.ds(wid * WORDS_PER_TILE, WORDS_PER_TILE)])


_densify = functools.partial(
    pl.kernel,
    out_type=jax.ShapeDtypeStruct((NC * N * D,), jnp.float32),
    mesh=_mesh,
    scratch_types=[
        pltpu.VMEM_SHARED((N * D,), jnp.float32),
        pltpu.VMEM((NCHUNK, CH), jnp.int32),
        pltpu.VMEM((NCHUNK, CH), jnp.int32),
        pltpu.VMEM((NCHUNK, CH), jnp.float32),
        pltpu.VMEM((NCHUNK, CH), jnp.int32),
        pltpu.VMEM((ZWORDS,), jnp.float32),
        pltpu.SemaphoreType.DMA,
    ],
)(_densify_body)


def _propagate_body(xw, ar, ac, av, out, acc, st_r, st_c, st_v, rbuf0, rbuf1,
                    gsem0, gsem1, ssem0, ssem1):
    c = lax.axis_index("c")
    s = lax.axis_index("s")
    wid = c * NS + s

    _zero_fill_rows(rbuf0, CH)
    for q in range(RPT // CH):
        pltpu.sync_copy(rbuf0, acc.at[pl.ds(s * RPT + q * CH, CH)])
    plsc.subcore_barrier()

    def scale(buf, j):
        def row_grp(g, _):
            v = st_v[j, pl.ds(g * 16, 16)]
            for i in range(16):
                w = v[i]
                for q in range(D // 16):
                    buf[g * 16 + i, pl.ds(q * 16, 16)] = (
                        buf[g * 16 + i, pl.ds(q * 16, 16)] * w)
            return 0

        lax.fori_loop(0, CH // 16, row_grp, 0)

    for blk in range(NBLK):
        pltpu.sync_copy(ar.at[wid, pl.ds(blk * BCH, BCH)], st_r)
        pltpu.sync_copy(ac.at[wid, pl.ds(blk * BCH, BCH)], st_c)
        pltpu.sync_copy(av.at[wid, pl.ds(blk * BCH, BCH)], st_v)

        pltpu.async_copy(xw.at[st_c.at[0]], rbuf0, gsem0)

        def pair_body(t, _):
            e = 2 * t
            o = 2 * t + 1
            pltpu.make_async_copy(xw.at[st_c.at[e]], rbuf0, gsem0).wait()
            pltpu.async_copy(xw.at[st_c.at[o]], rbuf1, gsem1)
            scale(rbuf0, e)
            pltpu.sync_copy(rbuf0, acc.at[st_r.at[e]], add=True)

            @pl.when(t < BCH // 2 - 1)
            def _():
                pltpu.async_copy(xw.at[st_c.at[e + 2]], rbuf0, gsem0)

            pltpu.make_async_copy(xw.at[st_c.at[o]], rbuf1, gsem1).wait()
            scale(rbuf1, o)
            pltpu.sync_copy(rbuf1, acc.at[st_r.at[o]], add=True)
            return 0

        lax.fori_loop(0, BCH // 2, pair_body, 0)

    plsc.subcore_barrier()

    pltpu.sync_copy(acc.at[pl.ds(s * RPT, RPT)],
                    out.at[c, pl.ds(s * RPT, RPT)])


_propagate = functools.partial(
    pl.kernel,
    out_type=jax.ShapeDtypeStruct((NC, N_PAD, D), jnp.float32),
    mesh=_mesh,
    scratch_types=[
        pltpu.VMEM_SHARED((N_PAD, D), jnp.float32),
        pltpu.VMEM((BCH, CH), jnp.int32),
        pltpu.VMEM((BCH, CH), jnp.int32),
        pltpu.VMEM((BCH, CH), jnp.float32),
        pltpu.VMEM((CH, D), jnp.float32),
        pltpu.VMEM((CH, D), jnp.float32),
        pltpu.SemaphoreType.DMA,
        pltpu.SemaphoreType.DMA,
        pltpu.SemaphoreType.DMA,
        pltpu.SemaphoreType.DMA,
    ],
)(_propagate_body)


def _matmul_body(xd_ref, w_ref, o_ref):
    o_ref[...] = jnp.dot(xd_ref[0] + xd_ref[1], w_ref[...],
                         preferred_element_type=jnp.float32)


def _finish_body(a_ref, b_ref, o_ref):
    o_ref[...] = jnp.maximum(a_ref[0] + a_ref[1] + b_ref[...], 0.0)


_BLK = 400


def _pad_split(a, pad):
    a = jnp.concatenate([a.reshape(NW, EPW), pad.astype(a.dtype)], axis=1)
    return a.reshape(NW, NCHUNK, CH)


def kernel(x_rows, x_cols, x_vals, adj_rows, adj_cols, adj_vals, W, b):
    wj = jnp.arange(NW, dtype=jnp.int32)[:, None]
    pj = jnp.arange(PAD, dtype=jnp.int32)[None, :]
    zpad = jnp.zeros((NW, PAD), jnp.float32)
    xr = _pad_split(x_rows, (wj * 37 + pj * 53) % N)
    xc = _pad_split(x_cols, (wj * 5 + pj) % D)
    xv = _pad_split(x_vals, zpad)
    ar = _pad_split(adj_rows, N + (wj * 7 + pj) % (N_PAD - N))
    ac = _pad_split(adj_cols, (wj * 131 + pj * 97) % N)
    av = _pad_split(adj_vals, zpad)

    xd = _densify(xr, xc, xv).reshape(NC, N, D)

    xw = pl.pallas_call(
        _matmul_body,
        grid=(N // _BLK,),
        in_specs=[
            pl.BlockSpec((NC, _BLK, D), lambda i: (0, i, 0)),
            pl.BlockSpec((D, D), lambda i: (0, 0)),
        ],
        out_specs=pl.BlockSpec((_BLK, D), lambda i: (i, 0)),
        out_shape=jax.ShapeDtypeStruct((N, D), jnp.float32),
    )(xd, W)

    ax = _propagate(xw, ar, ac, av)

    out = pl.pallas_call(
        _finish_body,
        grid=(N // _BLK,),
        in_specs=[
            pl.BlockSpec((NC, _BLK, D), lambda i: (0, i, 0)),
            pl.BlockSpec((1, D), lambda i: (0, 0)),
        ],
        out_specs=pl.BlockSpec((_BLK, D), lambda i: (i, 0)),
        out_shape=jax.ShapeDtypeStruct((N, D), jnp.float32),
    )(ax, b.reshape(1, D))

    return out
